# Initial kernel scaffold; baseline (speedup 1.0000x reference)
import jax
import jax.numpy as jnp
from jax.experimental import pallas as pl


def kernel(h, pos, vel, acc, crowd, mask, idex, hist, e_w1, e_b1, e_w2, e_b2, n_w, n_b, ln_g, ln_b):
    raise NotImplementedError("write your pallas kernel here")


# speedup vs baseline: 9.7370x; 9.7370x over previous
import functools

import jax
import jax.numpy as jnp
from jax import lax
from jax.experimental import pallas as pl
from jax.experimental.pallas import tpu as pltpu
from jax.experimental.pallas import tpu_sc as plsc

NC, NS, LANES = 2, 16, 16
NW = NC * NS


def _pre_body(h_ref, w_ref, b_ref, o_ref):
    o_ref[...] = (
        jnp.dot(h_ref[...], w_ref[...], preferred_element_type=jnp.float32)
        + b_ref[...]
    )


def _project(h2, w1h, b1):
    rows, d = h2.shape
    blk = 1024
    return pl.pallas_call(
        _pre_body,
        grid=(rows // blk,),
        in_specs=[
            pl.BlockSpec((blk, d), lambda i: (i, 0)),
            pl.BlockSpec((d, d), lambda i: (0, 0)),
            pl.BlockSpec((1, d), lambda i: (0, 0)),
        ],
        out_specs=pl.BlockSpec((blk, d), lambda i: (i, 0)),
        out_shape=jax.ShapeDtypeStruct((rows, d), jnp.float32),
    )(h2, w1h, b1)


def _make_gather(n_nodes, n_batches, n_edges, d, gw):
    e_tile = n_edges // NW
    edges_per_batch = n_edges // n_batches
    G = 128

    mesh = plsc.VectorSubcoreMesh(
        core_axis_name="c", subcore_axis_name="s", num_cores=NC,
        num_subcores=NS)

    def body(pre_hbm, geo_hbm, idx_hbm, epre_hbm, egeo_hbm,
             idxv, fidxv, rows_v, geo_tab, geo_buf, sem1):
        wid = lax.axis_index("s") * NC + lax.axis_index("c")
        base = wid * e_tile
        boff = (base // edges_per_batch) * n_nodes
        pltpu.sync_copy(geo_hbm.at[pl.ds(boff * gw, n_nodes * gw)], geo_tab)

        def chunk(ci, _):
            eb = base + ci * G
            pltpu.sync_copy(idx_hbm.at[pl.ds(eb, G)], idxv)
            for j in range(G // LANES):
                sl = pl.ds(j * LANES, LANES)
                ids = idxv[sl]
                fidxv[sl] = ids + boff
                gids = ids * gw
                for c in range(gw):
                    geo_buf[c, sl] = plsc.load_gather(geo_tab, [gids + c])
            cp1 = pltpu.async_copy(pre_hbm.at[fidxv], rows_v, sem1)
            cp1.wait()
            pltpu.sync_copy(rows_v, epre_hbm.at[pl.ds(eb, G)])
            pltpu.sync_copy(geo_buf, egeo_hbm.at[:, pl.ds(eb, G)])
            return 0

        lax.fori_loop(0, e_tile // G, chunk, 0)

    return pl.kernel(
        body,
        out_type=(
            jax.ShapeDtypeStruct((n_edges, d), jnp.float32),
            jax.ShapeDtypeStruct((gw, n_edges), jnp.float32),
        ),
        mesh=mesh,
        scratch_types=[
            pltpu.VMEM((G,), jnp.int32),
            pltpu.VMEM((G,), jnp.int32),
            pltpu.VMEM((G, d), jnp.float32),
            pltpu.VMEM((n_nodes * gw,), jnp.float32),
            pltpu.VMEM((gw, G), jnp.float32),
            pltpu.SemaphoreType.DMA,
        ],
        compiler_params=pltpu.CompilerParams(needs_layout_passes=False),
    )


def _node_body(ep_ref, eg_ref, pos_ref, vel_ref, mask_ref, h_ref, crowd_ref,
               wg_ref, w2_ref, b2_ref, nw1_ref, nw2_ref, nw3_ref, nb_ref,
               lng_ref, lnb_ref, o_ref):
    blk, k, d = ep_ref.shape
    ep = ep_ref[...]
    eg = eg_ref[...]
    pos = pos_ref[...]
    vel = vel_ref[...]
    mask = mask_ref[...]

    rpx = eg[0] - pos[:, 0:1]
    rpy = eg[1] - pos[:, 1:2]
    dist = jnp.sqrt(rpx * rpx + rpy * rpy + 1e-12) + 1e-6
    dvx = vel[:, 0:1] - eg[2]
    dvy = vel[:, 1:2] - eg[3]
    rs = jnp.sqrt(dvx * dvx + dvy * dvy + 1e-12)

    geof = jnp.stack([rpx, rpy, dist, rs], axis=-1)
    geomvec = jnp.dot(
        geof.reshape(blk * k, 4), wg_ref[...],
        preferred_element_type=jnp.float32).reshape(blk, k, d)

    eh = jnp.maximum(ep + geomvec, 0.0)
    s = jnp.sum(eh * mask[:, :, None], axis=1)
    msum = jnp.sum(mask, axis=1, keepdims=True)

    agg = (jnp.dot(s, w2_ref[...], preferred_element_type=jnp.float32)
           + msum * b2_ref[...]) / (msum + 1e-6)

    c = crowd_ref[...]
    mu = jnp.mean(c, axis=-1, keepdims=True)
    var = jnp.mean((c - mu) ** 2, axis=-1, keepdims=True)
    c1 = (c - mu) / jnp.sqrt(var + 1e-5) * lng_ref[...] + lnb_ref[...]

    node = (jnp.dot(h_ref[...], nw1_ref[...], preferred_element_type=jnp.float32)
            + jnp.dot(agg, nw2_ref[...], preferred_element_type=jnp.float32)
            + jnp.dot(c1, nw3_ref[...], preferred_element_type=jnp.float32)
            + nb_ref[...])
    o_ref[...] = jnp.maximum(node, 0.0)


def _node_stage(ep3, eg3, pos2, vel2, mask2, h2, crowd2,
                wg, w2, b2, nw1, nw2, nw3, nb, lng, lnb):
    rows, k, d = ep3.shape
    gw = eg3.shape[0]
    cw = crowd2.shape[-1]
    out_w = nw1.shape[1]
    blk = 128
    grid = (rows // blk,)
    full = lambda shape: pl.BlockSpec(shape, lambda i: tuple(0 for _ in shape))
    return pl.pallas_call(
        _node_body,
        grid=grid,
        in_specs=[
            pl.BlockSpec((blk, k, d), lambda i: (i, 0, 0)),
            pl.BlockSpec((gw, blk, k), lambda i: (0, i, 0)),
            pl.BlockSpec((blk, 2), lambda i: (i, 0)),
            pl.BlockSpec((blk, 2), lambda i: (i, 0)),
            pl.BlockSpec((blk, k), lambda i: (i, 0)),
            pl.BlockSpec((blk, d), lambda i: (i, 0)),
            pl.BlockSpec((blk, cw), lambda i: (i, 0)),
            full((4, d)),
            full((d, out_w)),
            full((1, out_w)),
            full((d, out_w)),
            full((d, out_w)),
            full((cw, out_w)),
            full((1, out_w)),
            full((1, cw)),
            full((1, cw)),
        ],
        out_specs=pl.BlockSpec((blk, out_w), lambda i: (i, 0)),
        out_shape=jax.ShapeDtypeStruct((rows, out_w), jnp.float32),
    )(ep3, eg3, pos2, vel2, mask2, h2, crowd2,
      wg, w2, b2, nw1, nw2, nw3, nb, lng, lnb)


def kernel(h, pos, vel, acc, crowd, mask, idex, hist,
           e_w1, e_b1, e_w2, e_b2, n_w, n_b, ln_g, ln_b):
    B, N, D = h.shape
    K = idex.shape[-1]
    OUT = e_w2.shape[1]
    CW = crowd.shape[-1]
    n_edges = B * N * K

    h2 = h.reshape(B * N, D)
    pre = _project(h2, e_w1[:D], e_b1.reshape(1, OUT))

    geo_t = jnp.concatenate([pos, vel], axis=-1).reshape(B * N * 4)
    idx_flat = idex.reshape(n_edges)

    epre, egeo = _make_gather(N, B, n_edges, D, 4)(pre, geo_t, idx_flat)

    out = _node_stage(
        epre.reshape(B * N, K, D),
        egeo.reshape(4, B * N, K),
        pos.reshape(B * N, 2),
        vel.reshape(B * N, 2),
        mask.reshape(B * N, K),
        h2,
        crowd.reshape(B * N, CW),
        e_w1[D:D + 4],
        e_w2,
        e_b2.reshape(1, OUT),
        n_w[:D],
        n_w[D:2 * D],
        n_w[2 * D:],
        n_b.reshape(1, OUT),
        ln_g.reshape(1, CW),
        ln_b.reshape(1, CW),
    )
    return out.reshape(B, N, OUT)


# speedup vs baseline: 11.7997x; 1.2118x over previous
import functools

import jax
import jax.numpy as jnp
from jax import lax
from jax.experimental import pallas as pl
from jax.experimental.pallas import tpu as pltpu
from jax.experimental.pallas import tpu_sc as plsc

NC, NS, LANES = 2, 16, 16
NW = NC * NS


def _pre_body(h_ref, w_ref, b_ref, o_ref):
    o_ref[...] = (
        jnp.dot(h_ref[...], w_ref[...], preferred_element_type=jnp.float32)
        + b_ref[...]
    )


def _project(h2, w1h, b1):
    rows, d = h2.shape
    blk = 1024
    return pl.pallas_call(
        _pre_body,
        grid=(rows // blk,),
        in_specs=[
            pl.BlockSpec((blk, d), lambda i: (i, 0)),
            pl.BlockSpec((d, d), lambda i: (0, 0)),
            pl.BlockSpec((1, d), lambda i: (0, 0)),
        ],
        out_specs=pl.BlockSpec((blk, d), lambda i: (i, 0)),
        out_shape=jax.ShapeDtypeStruct((rows, d), jnp.float32),
    )(h2, w1h, b1)


def _make_gather(n_nodes, n_batches, n_edges, d, gw):
    e_tile = n_edges // NW
    edges_per_batch = n_edges // n_batches
    G = 128

    mesh = plsc.VectorSubcoreMesh(
        core_axis_name="c", subcore_axis_name="s", num_cores=NC,
        num_subcores=NS)

    steps = e_tile // G

    def body(pre_hbm, geo_hbm, idx_hbm, epre_hbm, egeo_hbm,
             idx_all, fidx_all, geo_tab, geo_all, rows0, rows1,
             sem_g0, sem_g1, sem_w0, sem_w1):
        wid = lax.axis_index("s") * NC + lax.axis_index("c")
        base = wid * e_tile
        boff = (base // edges_per_batch) * n_nodes
        pltpu.sync_copy(idx_hbm.at[pl.ds(base, e_tile)], idx_all)
        pltpu.sync_copy(geo_hbm.at[pl.ds(boff * gw, n_nodes * gw)], geo_tab)

        def fill(i, _):
            sl = pl.ds(i * LANES, LANES)
            ids = idx_all[sl]
            fidx_all[sl] = ids + boff
            gids = ids * gw
            for c in range(gw):
                geo_all[c, sl] = plsc.load_gather(geo_tab, [gids + c])
            return 0

        lax.fori_loop(0, e_tile // LANES, fill, 0)

        rows = (rows0, rows1)
        sem_g = (sem_g0, sem_g1)
        sem_w = (sem_w0, sem_w1)

        def gather(c, b):
            pltpu.async_copy(
                pre_hbm.at[fidx_all.at[pl.ds(c * G, G)]], rows[b], sem_g[b])

        def wait_gather(c, b):
            pltpu.make_async_copy(
                pre_hbm.at[fidx_all.at[pl.ds(c * G, G)]], rows[b],
                sem_g[b]).wait()

        def write(c, b):
            pltpu.async_copy(
                rows[b], epre_hbm.at[pl.ds(base + c * G, G)], sem_w[b])

        def wait_write(c, b):
            pltpu.make_async_copy(
                rows[b], epre_hbm.at[pl.ds(base + c * G, G)],
                sem_w[b]).wait()

        gather(0, 0)
        gather(1, 1)
        wait_gather(0, 0)
        write(0, 0)

        def pair(p, _):
            for b in (0, 1):
                c = 2 * p + b
                wait_write(c - 2, b)
                gather(c, b)
                wait_gather(c - 1, 1 - b)
                write(c - 1, 1 - b)
            return 0

        lax.fori_loop(1, steps // 2, pair, 0)

        last = steps - 1
        wait_gather(last, last % 2)
        write(last, last % 2)
        wait_write(last - 1, (last - 1) % 2)
        wait_write(last, last % 2)
        pltpu.sync_copy(geo_all, egeo_hbm.at[:, pl.ds(base, e_tile)])

    return pl.kernel(
        body,
        out_type=(
            jax.ShapeDtypeStruct((n_edges, d), jnp.float32),
            jax.ShapeDtypeStruct((gw, n_edges), jnp.float32),
        ),
        mesh=mesh,
        scratch_types=[
            pltpu.VMEM((e_tile,), jnp.int32),
            pltpu.VMEM((e_tile,), jnp.int32),
            pltpu.VMEM((n_nodes * gw,), jnp.float32),
            pltpu.VMEM((gw, e_tile), jnp.float32),
            pltpu.VMEM((G, d), jnp.float32),
            pltpu.VMEM((G, d), jnp.float32),
            pltpu.SemaphoreType.DMA,
            pltpu.SemaphoreType.DMA,
            pltpu.SemaphoreType.DMA,
            pltpu.SemaphoreType.DMA,
        ],
        compiler_params=pltpu.CompilerParams(needs_layout_passes=False),
    )


def _node_body(ep_ref, eg_ref, pos_ref, vel_ref, mask_ref, h_ref, crowd_ref,
               wg_ref, w2_ref, b2_ref, nw1_ref, nw2_ref, nw3_ref, nb_ref,
               lng_ref, lnb_ref, o_ref):
    blk, k, d = ep_ref.shape
    ep = ep_ref[...]
    eg = eg_ref[...]
    pos = pos_ref[...]
    vel = vel_ref[...]
    mask = mask_ref[...]

    rpx = eg[0] - pos[:, 0:1]
    rpy = eg[1] - pos[:, 1:2]
    dist = jnp.sqrt(rpx * rpx + rpy * rpy + 1e-12) + 1e-6
    dvx = vel[:, 0:1] - eg[2]
    dvy = vel[:, 1:2] - eg[3]
    rs = jnp.sqrt(dvx * dvx + dvy * dvy + 1e-12)

    geof = jnp.stack([rpx, rpy, dist, rs], axis=-1)
    geomvec = jnp.dot(
        geof.reshape(blk * k, 4), wg_ref[...],
        preferred_element_type=jnp.float32).reshape(blk, k, d)

    eh = jnp.maximum(ep + geomvec, 0.0)
    s = jnp.sum(eh * mask[:, :, None], axis=1)
    msum = jnp.sum(mask, axis=1, keepdims=True)

    agg = (jnp.dot(s, w2_ref[...], preferred_element_type=jnp.float32)
           + msum * b2_ref[...]) / (msum + 1e-6)

    c = crowd_ref[...]
    mu = jnp.mean(c, axis=-1, keepdims=True)
    var = jnp.mean((c - mu) ** 2, axis=-1, keepdims=True)
    c1 = (c - mu) / jnp.sqrt(var + 1e-5) * lng_ref[...] + lnb_ref[...]

    node = (jnp.dot(h_ref[...], nw1_ref[...], preferred_element_type=jnp.float32)
            + jnp.dot(agg, nw2_ref[...], preferred_element_type=jnp.float32)
            + jnp.dot(c1, nw3_ref[...], preferred_element_type=jnp.float32)
            + nb_ref[...])
    o_ref[...] = jnp.maximum(node, 0.0)


def _node_stage(ep3, eg3, pos2, vel2, mask2, h2, crowd2,
                wg, w2, b2, nw1, nw2, nw3, nb, lng, lnb):
    rows, k, d = ep3.shape
    gw = eg3.shape[0]
    cw = crowd2.shape[-1]
    out_w = nw1.shape[1]
    blk = 128
    grid = (rows // blk,)
    full = lambda shape: pl.BlockSpec(shape, lambda i: tuple(0 for _ in shape))
    return pl.pallas_call(
        _node_body,
        grid=grid,
        in_specs=[
            pl.BlockSpec((blk, k, d), lambda i: (i, 0, 0)),
            pl.BlockSpec((gw, blk, k), lambda i: (0, i, 0)),
            pl.BlockSpec((blk, 2), lambda i: (i, 0)),
            pl.BlockSpec((blk, 2), lambda i: (i, 0)),
            pl.BlockSpec((blk, k), lambda i: (i, 0)),
            pl.BlockSpec((blk, d), lambda i: (i, 0)),
            pl.BlockSpec((blk, cw), lambda i: (i, 0)),
            full((4, d)),
            full((d, out_w)),
            full((1, out_w)),
            full((d, out_w)),
            full((d, out_w)),
            full((cw, out_w)),
            full((1, out_w)),
            full((1, cw)),
            full((1, cw)),
        ],
        out_specs=pl.BlockSpec((blk, out_w), lambda i: (i, 0)),
        out_shape=jax.ShapeDtypeStruct((rows, out_w), jnp.float32),
    )(ep3, eg3, pos2, vel2, mask2, h2, crowd2,
      wg, w2, b2, nw1, nw2, nw3, nb, lng, lnb)


def kernel(h, pos, vel, acc, crowd, mask, idex, hist,
           e_w1, e_b1, e_w2, e_b2, n_w, n_b, ln_g, ln_b):
    B, N, D = h.shape
    K = idex.shape[-1]
    OUT = e_w2.shape[1]
    CW = crowd.shape[-1]
    n_edges = B * N * K

    h2 = h.reshape(B * N, D)
    pre = _project(h2, e_w1[:D], e_b1.reshape(1, OUT))

    geo_t = jnp.concatenate([pos, vel], axis=-1).reshape(B * N * 4)
    idx_flat = idex.reshape(n_edges)

    epre, egeo = _make_gather(N, B, n_edges, D, 4)(pre, geo_t, idx_flat)

    out = _node_stage(
        epre.reshape(B * N, K, D),
        egeo.reshape(4, B * N, K),
        pos.reshape(B * N, 2),
        vel.reshape(B * N, 2),
        mask.reshape(B * N, K),
        h2,
        crowd.reshape(B * N, CW),
        e_w1[D:D + 4],
        e_w2,
        e_b2.reshape(1, OUT),
        n_w[:D],
        n_w[D:2 * D],
        n_w[2 * D:],
        n_b.reshape(1, OUT),
        ln_g.reshape(1, CW),
        ln_b.reshape(1, CW),
    )
    return out.reshape(B, N, OUT)
